# baseline (speedup 1.0000x reference)
import jax
import jax.numpy as jnp
import numpy as np
from jax import lax
from jax.experimental import pallas as pl
from jax.experimental.pallas import tpu as pltpu
from jax.experimental.pallas import tpu_sc as plsc

N = 10000
NP = 10240
D = 128
H = 64
B = 64
K = 30
E = 320000
CH = 128
CPT = 79
EPT = CH * CPT
EP = EPT * 32
EPA = EP + 256
RPW = NP // 32
GP = 2048

_F32 = jnp.float32


def _dot(a, b):
    return lax.dot_general(a, b, (((1,), (0,)), ((), ())),
                           preferred_element_type=_F32)


_SC_CACHE = {}


def _mesh():
    if "mesh" not in _SC_CACHE:
        _SC_CACHE["mesh"] = plsc.VectorSubcoreMesh(
            core_axis_name="c", subcore_axis_name="s")
    return _SC_CACHE["mesh"]


def _floop(n, body_fn):
    lax.fori_loop(jnp.int32(0), jnp.int32(n),
                  lambda i, c: (body_fn(i), c)[1], jnp.int32(0))


def _make_fold_body(w):
    ng = CH // 16

    def body(table, srcs, dsts, bnd, acc_out, idx_s, dst_v, bnd_v, rows, acc, sem):
        c = lax.axis_index("c")
        s = lax.axis_index("s")
        wid = c * jnp.int32(16) + s
        iota = lax.iota(jnp.int32, 16)

        pltpu.sync_copy(bnd, bnd_v)

        def _extract(vec, lane):
            return jnp.sum(jnp.where(iota == lane, vec, 0), dtype=jnp.int32)

        def _bnd_at(i):
            grp = bnd_v[pl.ds((i // jnp.int32(16)) * jnp.int32(16), 16)]
            return _extract(grp, i % jnp.int32(16))

        e0 = _bnd_at(wid)
        e1 = _bnd_at(wid + jnp.int32(1))
        row0 = wid * jnp.int32(RPW)

        def _zero(r):
            for j in range(w // 16):
                acc[r, pl.ds(j * 16, 16)] = jnp.zeros((16,), _F32)

        _floop(RPW + 16, _zero)

        c0 = pl.multiple_of(e0 - (e0 % jnp.int32(8)), 8)
        nch = (e1 - c0 + jnp.int32(CH - 1)) // jnp.int32(CH)

        def _chunk(k, carry):
            off = pl.multiple_of(c0 + k * jnp.int32(CH), 8)
            pltpu.sync_copy(srcs.at[pl.ds(off, CH)], idx_s)
            pltpu.sync_copy(dsts.at[pl.ds(off, CH)], dst_v)
            pltpu.async_copy(table.at[idx_s], rows, sem).wait()
            for g in range(ng):
                grp = dst_v[pl.ds(g * 16, 16)]
                for l in range(16):
                    e_glob = off + jnp.int32(g * 16 + l)
                    d = _extract(grp, jnp.int32(l))
                    valid = (e_glob >= e0) & (e_glob < e1)
                    dl = jnp.where(valid, d - row0, jnp.int32(RPW))
                    for j in range(w // 16):
                        sl = pl.ds(j * 16, 16)
                        acc[dl, sl] = acc[dl, sl] + rows[g * 16 + l, sl]
            return carry

        lax.fori_loop(jnp.int32(0), nch, _chunk, jnp.int32(0))
        pltpu.sync_copy(acc.at[pl.ds(jnp.int32(0), RPW)],
                        acc_out.at[pl.ds(row0, RPW)])

    return body


def _segsum_sorted(table, srcs, dsts, bnd, w):
    key = ("fold", w)
    if key not in _SC_CACHE:
        _SC_CACHE[key] = pl.kernel(
            _make_fold_body(w),
            out_type=jax.ShapeDtypeStruct((NP, w), _F32),
            mesh=_mesh(),
            compiler_params=pltpu.CompilerParams(
                use_tc_tiling_on_sc=False, needs_layout_passes=False),
            scratch_types=[
                pltpu.VMEM((CH,), jnp.int32),
                pltpu.VMEM((CH,), jnp.int32),
                pltpu.VMEM((48,), jnp.int32),
                pltpu.VMEM((CH, w), _F32),
                pltpu.VMEM((RPW + 16, w), _F32),
                pltpu.SemaphoreType.DMA,
            ],
        )
    return _SC_CACHE[key](table, srcs, dsts, bnd)


def _gather_body(h3, idxh, out, idxv, rows, sem):
    wid = lax.axis_index("c") * jnp.int32(16) + lax.axis_index("s")
    base = wid * jnp.int32(GP // 32)
    pltpu.sync_copy(idxh.at[pl.ds(base, GP // 32)], idxv)
    pltpu.async_copy(h3.at[idxv], rows, sem).wait()
    pltpu.sync_copy(rows, out.at[pl.ds(base, GP // 32)])


def _gather_rows(h3, idx_flat):
    if "gather" not in _SC_CACHE:
        _SC_CACHE["gather"] = pl.kernel(
            _gather_body,
            out_type=jax.ShapeDtypeStruct((GP, H), _F32),
            mesh=_mesh(),
            compiler_params=pltpu.CompilerParams(use_tc_tiling_on_sc=False),
            scratch_types=[
                pltpu.VMEM((GP // 32,), jnp.int32),
                pltpu.VMEM((GP // 32, H), _F32),
                pltpu.SemaphoreType.DMA,
            ],
        )
    return _SC_CACHE["gather"](h3, idx_flat)


def _topk_body(h_ref, b_ref, idx_ref, mat_ref):
    biota = lax.broadcasted_iota(jnp.int32, (NP, B), 1)
    riota = lax.broadcasted_iota(jnp.int32, (NP, B), 0)
    last = h_ref[...][:, H - 1:H]
    neg = jnp.float32(-jnp.inf)
    mat_ref[...] = jnp.where(b_ref[...] == biota,
                             jnp.broadcast_to(last, (NP, B)), neg)

    def step(k, carry):
        mat = mat_ref[...]
        m = jnp.max(mat, axis=0, keepdims=True)
        cand = jnp.where(mat == m, riota, jnp.int32(1 << 30))
        w = jnp.min(cand, axis=0, keepdims=True)
        idx_ref[pl.ds(k, 1), :] = jnp.where(m >= 0.0, w, jnp.int32(NP - 1))
        mat_ref[...] = jnp.where(riota == w, neg, mat)
        return carry

    lax.fori_loop(jnp.int32(0), jnp.int32(K), step, jnp.int32(0))


def _topk(h3, batch2d):
    return pl.pallas_call(
        _topk_body,
        out_shape=jax.ShapeDtypeStruct((K, B), jnp.int32),
        scratch_shapes=[pltpu.VMEM((NP, B), _F32)],
    )(h3, batch2d)


def _head_body(g_ref, wb_ref, cb_ref, l1_ref, b1_ref, l2_ref, b2_ref, o_ref):
    y = jnp.maximum(_dot(g_ref[...], wb_ref[...]) + cb_ref[...], 0.0)
    z = jnp.maximum(_dot(y, l1_ref[...]) + b1_ref[...], 0.0)
    o_ref[...] = _dot(z, l2_ref[...]) + b2_ref[...]


def _head(gflat, wbig, cbt, l1t, b1, l2t, b2):
    return pl.pallas_call(
        _head_body,
        out_shape=jax.ShapeDtypeStruct((B, 1), _F32),
    )(gflat, wbig, cbt, l1t, b1, l2t, b2)


def kernel(x, edge_index, batch, w1_l, b1_l, w1_r, w2_l, b2_l, w2_r,
           w3_l, b3_l, w3_r, conv_w, conv_b, lin1_w, lin1_b, lin2_w, lin2_b):
    x = x.astype(_F32)
    ei = edge_index.astype(jnp.int32)
    pad_e = jnp.full((EPA - E,), N, jnp.int32)
    src = jnp.concatenate([ei[0], pad_e])
    dst = jnp.concatenate([ei[1], pad_e])
    dsts, srcs = lax.sort((dst, src), num_keys=1, is_stable=True)
    starts = jnp.searchsorted(dsts, jnp.arange(N + 1, dtype=jnp.int32)
                              ).astype(jnp.int32)
    cnt_m = jnp.maximum((starts[1:] - starts[:N]).astype(_F32), 1.0)[:, None]
    bnd = jnp.pad(
        jnp.searchsorted(dsts, jnp.arange(33, dtype=jnp.int32) * RPW
                         ).astype(jnp.int32), (0, 15))
    batch_p = jnp.concatenate(
        [batch.astype(jnp.int32), jnp.full((NP - N,), B, jnp.int32)]
    ).reshape(NP, 1)

    def sage(h, wl, bl, wr):
        hp = jnp.pad(h, ((0, NP - h.shape[0]), (0, 0)))
        s = _segsum_sorted(hp, srcs, dsts, bnd, hp.shape[1])[:N]
        mean = s / cnt_m
        return jax.nn.relu(mean @ wl.T + bl + h @ wr.T)

    h1 = sage(x, w1_l, b1_l, w1_r)
    h2 = sage(h1, w2_l, b2_l, w2_r)
    h3 = sage(h2, w3_l, b3_l, w3_r)

    h3p = jnp.pad(h3, ((0, NP - N), (0, 0)))
    idxk = _topk(h3p, batch_p)
    idx_flat = jnp.concatenate(
        [idxk.T.reshape(B * K), jnp.full((GP - B * K,), NP - 1, jnp.int32)])
    g = _gather_rows(h3p, idx_flat)
    gflat = g[:B * K].reshape(B, K * H)

    wb = jnp.zeros((K, H, K - 2, 32), _F32)
    t_ar = jnp.arange(K - 2)
    for dt in range(3):
        wb = wb.at[t_ar + dt, :, t_ar, :].set(
            jnp.broadcast_to(conv_w[:, :, dt].T.astype(_F32), (K - 2, H, 32)))
    wbig = wb.reshape(K * H, (K - 2) * 32)
    cbt = jnp.tile(conv_b.astype(_F32), K - 2).reshape(1, (K - 2) * 32)
    l1t = (lin1_w.reshape(H, 32, K - 2).transpose(0, 2, 1)
           .reshape(H, (K - 2) * 32)).T.astype(_F32)
    b1h = lin1_b.reshape(1, H).astype(_F32)
    l2t = lin2_w.T.astype(_F32)
    b2h = lin2_b.reshape(1, 1).astype(_F32)

    return _head(gflat, wbig, cbt, l1t, b1h, l2t, b2h)
